# baseline (speedup 1.0000x reference)
import jax
import jax.numpy as jnp
from jax.experimental import pallas as pl


def _trim_kernel(x_ref, v_ref, m_ref, xo_ref, vo_ref, mo_ref):
    xo_ref[...] = x_ref[...]
    vo_ref[...] = v_ref[...]
    mo_ref[...] = m_ref[...] != 0.0


def kernel(x, v, mask):
    b, n, l = x.shape
    _, nv, _ = v.shape
    _, nm, _ = mask.shape
    xo, vo, mo = pl.pallas_call(
        _trim_kernel,
        grid=(b,),
        in_specs=[
            pl.BlockSpec((1, n, l), lambda i: (i, 0, 0)),
            pl.BlockSpec((1, nv, l), lambda i: (i, 0, 0)),
            pl.BlockSpec((1, nm, l), lambda i: (i, 0, 0)),
        ],
        out_specs=[
            pl.BlockSpec((1, n, l), lambda i: (i, 0, 0)),
            pl.BlockSpec((1, nv, l), lambda i: (i, 0, 0)),
            pl.BlockSpec((1, nm, l), lambda i: (i, 0, 0)),
        ],
        out_shape=[
            jax.ShapeDtypeStruct(x.shape, x.dtype),
            jax.ShapeDtypeStruct(v.shape, v.dtype),
            jax.ShapeDtypeStruct(mask.shape, jnp.bool_),
        ],
    )(x, v, mask)
    return (xo, vo, mo)
